# baseline (speedup 1.0000x reference)
import functools

import numpy as np
import jax
import jax.numpy as jnp
from jax import lax
from jax.experimental import pallas as pl
from jax.experimental.pallas import tpu as pltpu
from jax.experimental.pallas import tpu_sc as plsc

Q = 1024
K = 100000
D = 128
NN = 32
HIDDEN = 256

KQ_BLK = 128
KK_BLK = 2048
K_PAD = ((K + KK_BLK - 1) // KK_BLK) * KK_BLK

NF = 256
MQ_BLK = 256


def _d2_body(x_ref, xd_ref, out_ref):
    j = pl.program_id(0)
    xb = x_ref[...]
    db = xd_ref[...]
    x_sq = jnp.sum(xb * xb, axis=1, keepdims=True)
    d_sq = jnp.sum(db * db, axis=1)
    cross = jax.lax.dot_general(
        xb, db, (((1,), (1,)), ((), ())),
        preferred_element_type=jnp.float32,
        precision=jax.lax.Precision.DEFAULT)
    d2 = x_sq + d_sq[None, :] - 2.0 * cross
    col = j * KK_BLK + lax.broadcasted_iota(jnp.int32, (KQ_BLK, KK_BLK), 1)
    out_ref[...] = jnp.where(col < K, d2, jnp.float32(jnp.inf))


def _d2_matrix(x, X_pad):
    return pl.pallas_call(
        _d2_body,
        grid=(K_PAD // KK_BLK, Q // KQ_BLK),
        in_specs=[
            pl.BlockSpec((KQ_BLK, D), lambda j, i: (i, 0)),
            pl.BlockSpec((KK_BLK, D), lambda j, i: (j, 0)),
        ],
        out_specs=pl.BlockSpec((KQ_BLK, KK_BLK), lambda j, i: (i, j)),
        out_shape=jax.ShapeDtypeStruct((Q, K_PAD), jnp.float32),
    )(x, X_pad)


def _mlp_body(x_ref, g_ref, w1x_ref, w1g_ref, b1_ref, w2_ref, b2_ref,
              w3_ref, b3_ref, out_ref):
    hp = jax.lax.Precision.HIGHEST
    a1 = (jax.lax.dot_general(x_ref[...], w1x_ref[...], (((1,), (0,)), ((), ())),
                              preferred_element_type=jnp.float32, precision=hp)
          + jax.lax.dot_general(g_ref[...], w1g_ref[...], (((1,), (0,)), ((), ())),
                                preferred_element_type=jnp.float32, precision=hp)
          + b1_ref[...])
    h1 = jnp.tanh(a1)
    a2 = jax.lax.dot_general(h1, w2_ref[...], (((1,), (0,)), ((), ())),
                             preferred_element_type=jnp.float32, precision=hp) + b2_ref[...]
    h2 = jnp.tanh(a2)
    a3 = jax.lax.dot_general(h2, w3_ref[...], (((1,), (0,)), ((), ())),
                             preferred_element_type=jnp.float32, precision=hp) + b3_ref[0, 0]
    out_ref[...] = jax.nn.sigmoid(a3)


def _mlp(x, g, w1x_t, w1g_t, b1, w2_t, b2, w3_t, b3):
    gin = NN * NF
    return pl.pallas_call(
        _mlp_body,
        grid=(Q // MQ_BLK,),
        in_specs=[
            pl.BlockSpec((MQ_BLK, D), lambda i: (i, 0)),
            pl.BlockSpec((MQ_BLK, gin), lambda i: (i, 0)),
            pl.BlockSpec((D, HIDDEN), lambda i: (0, 0)),
            pl.BlockSpec((gin, HIDDEN), lambda i: (0, 0)),
            pl.BlockSpec((1, HIDDEN), lambda i: (0, 0)),
            pl.BlockSpec((HIDDEN, HIDDEN), lambda i: (0, 0)),
            pl.BlockSpec((1, HIDDEN), lambda i: (0, 0)),
            pl.BlockSpec((HIDDEN, 1), lambda i: (0, 0)),
            pl.BlockSpec((1, 1), lambda i: (0, 0)),
        ],
        out_specs=pl.BlockSpec((MQ_BLK, 1), lambda i: (i, 0)),
        out_shape=jax.ShapeDtypeStruct((Q, 1), jnp.float32),
    )(x, g, w1x_t, w1g_t, b1[None, :], w2_t, b2[None, :], w3_t, b3[None, :])


NSEL = NN + 1
CAP = 128
GRP = 8
VB = CAP + GRP * 16
VB16 = VB // 16
SPAD = 48
N_WORKERS = 32
QPW = Q // N_WORKERS
INF = np.float32(np.inf)


def _topk_gather(d2, xaug):
    mesh = plsc.VectorSubcoreMesh(core_axis_name="c", subcore_axis_name="s")

    @functools.partial(
        pl.kernel,
        out_type=jax.ShapeDtypeStruct((Q, NN, NF), jnp.float32),
        mesh=mesh,
        compiler_params=pltpu.CompilerParams(needs_layout_passes=False),
        scratch_types=[
            pltpu.VMEM((K_PAD,), jnp.float32),
            pltpu.VMEM((VB,), jnp.float32),
            pltpu.VMEM((VB,), jnp.int32),
            pltpu.VMEM((SPAD,), jnp.float32),
            pltpu.VMEM((SPAD,), jnp.int32),
            pltpu.VMEM((NN,), jnp.int32),
            pltpu.VMEM((NN, NF), jnp.float32),
            pltpu.SemaphoreType.DMA,
        ],
    )
    def sc_kernel(d2_hbm, xaug_hbm, g_hbm, row, vbuf, ibuf, topv, topi,
                  idxg, grows, sem):
        wid = lax.axis_index("s") * 2 + lax.axis_index("c")
        lane = lax.iota(jnp.int32, 16)

        def rebuild(count, t):
            del count, t

            def round_body(r, _):
                bestv = vbuf[pl.ds(0, 16)]
                for b in range(1, VB16):
                    bestv = jnp.minimum(bestv, vbuf[pl.ds(b * 16, 16)])
                mv = jnp.min(bestv)
                taken = np.int32(0)
                rr = jnp.full((16,), r, jnp.int32)
                for b in range(VB16):
                    v = vbuf[pl.ds(b * 16, 16)]
                    m = v == mv
                    mi = m.astype(jnp.int32)
                    pre = plsc.cumsum(mi)
                    sel = m & (pre == 1) & (taken == 0)
                    iv = ibuf[pl.ds(b * 16, 16)]
                    plsc.store_scatter(topv, [rr], v, mask=sel)
                    plsc.store_scatter(topi, [rr], iv, mask=sel)
                    vbuf[pl.ds(b * 16, 16)] = jnp.where(sel, INF, v)
                    taken = taken + jnp.sum(mi)
                return mv

            t33 = lax.fori_loop(0, NSEL, round_body, np.float32(0.0))
            for b in range(VB16):
                pos = lane + b * 16
                if b * 16 < SPAD:
                    tv = topv[pl.ds(b * 16, 16)]
                    ti = topi[pl.ds(b * 16, 16)]
                    vbuf[pl.ds(b * 16, 16)] = jnp.where(pos < NSEL, tv, INF)
                    ibuf[pl.ds(b * 16, 16)] = ti
                else:
                    vbuf[pl.ds(b * 16, 16)] = jnp.full((16,), INF)
            return jnp.full((16,), NSEL, jnp.int32), jnp.full((16,), t33)

        def group_body(gi, carry):
            count, t = carry
            for u in range(GRP):
                i = gi * GRP + u
                v = row[pl.ds(i * 16, 16)]
                m = v < t
                pre = plsc.cumsum(m.astype(jnp.int32))
                addr = count + pre - 1
                plsc.store_scatter(vbuf, [addr], v, mask=m)
                plsc.store_scatter(ibuf, [addr], lane + i * 16, mask=m)
                count = count + plsc.all_reduce_population_count(m)
            return lax.cond(jnp.max(count) >= CAP, rebuild,
                            lambda c, tt: (c, tt), count, t)

        def process_query(j, _):
            q = wid * QPW + j
            pltpu.sync_copy(d2_hbm.at[q], row)
            for b in range(VB16):
                vbuf[pl.ds(b * 16, 16)] = jnp.full((16,), INF)
            count, t = lax.fori_loop(
                0, K_PAD // (16 * GRP), group_body,
                (jnp.zeros((16,), jnp.int32), jnp.full((16,), INF)))
            rebuild(count, t)
            idxg[pl.ds(0, 16)] = topi[pl.ds(1, 16)]
            idxg[pl.ds(16, 16)] = topi[pl.ds(17, 16)]
            pltpu.async_copy(xaug_hbm.at[idxg], grows, sem).wait()
            pltpu.sync_copy(grows, g_hbm.at[q])
            return 0

        lax.fori_loop(0, QPW, process_query, 0)

    return sc_kernel(d2, xaug)


def kernel(x, X_data, y, W1, b1, W2, b2, W3, b3):
    X_pad = jnp.pad(X_data, ((0, K_PAD - K), (0, 0)))
    d2 = _d2_matrix(x, X_pad)

    Xaug = jnp.concatenate(
        [X_pad, jnp.pad(y, (0, K_PAD - K))[:, None],
         jnp.zeros((K_PAD, NF - D - 1), jnp.float32)], axis=1)

    g = _topk_gather(d2, Xaug).reshape(Q, NN * NF)

    w1x_t = W1[:, :D].T
    w1n = W1[:, D:].reshape(HIDDEN, NN, D + 1)
    w1g = jnp.concatenate(
        [w1n, jnp.zeros((HIDDEN, NN, NF - D - 1), jnp.float32)], axis=2)
    w1g_t = w1g.reshape(HIDDEN, NN * NF).T

    return _mlp(x, g, w1x_t, w1g_t, b1, W2.T, b2, W3.T, b3)


# speedup vs baseline: 1.9922x; 1.9922x over previous
import functools

import numpy as np
import jax
import jax.numpy as jnp
from jax import lax
from jax.experimental import pallas as pl
from jax.experimental.pallas import tpu as pltpu
from jax.experimental.pallas import tpu_sc as plsc

Q = 1024
K = 100000
D = 128
NN = 32
HIDDEN = 256

KQ_BLK = 128
KK_BLK = 2048
K_PAD = ((K + KK_BLK - 1) // KK_BLK) * KK_BLK

NF = 256
MQ_BLK = 256


GSZ = 128
NG = K_PAD // GSZ
GPB = KK_BLK // GSZ
NT = K_PAD // KK_BLK


def _d2_body(x_ref, xd_ref, out_ref, bm_ref):
    j = pl.program_id(0)
    xb = x_ref[...]
    db = xd_ref[...]
    x_sq = jnp.sum(xb * xb, axis=1, keepdims=True)
    d_sq = jnp.sum(db * db, axis=1)
    cross = jax.lax.dot_general(
        xb, db, (((1,), (1,)), ((), ())),
        preferred_element_type=jnp.float32,
        precision=jax.lax.Precision.DEFAULT)
    d2 = x_sq + d_sq[None, :] - 2.0 * cross
    col = j * KK_BLK + lax.broadcasted_iota(jnp.int32, (KQ_BLK, KK_BLK), 1)
    d2 = jnp.where(col < K, d2, jnp.float32(jnp.inf))
    d2g = d2.reshape(KQ_BLK, GPB, GSZ)
    out_ref[...] = d2g
    bm = jnp.min(d2g, axis=2)
    bm_ref[...] = jnp.concatenate(
        [bm, jnp.full((KQ_BLK, 128 - GPB), jnp.inf, jnp.float32)], axis=1)


def _d2_matrix(x, X_pad):
    return pl.pallas_call(
        _d2_body,
        grid=(K_PAD // KK_BLK, Q // KQ_BLK),
        in_specs=[
            pl.BlockSpec((KQ_BLK, D), lambda j, i: (i, 0)),
            pl.BlockSpec((KK_BLK, D), lambda j, i: (j, 0)),
        ],
        out_specs=[
            pl.BlockSpec((KQ_BLK, GPB, GSZ), lambda j, i: (i, j, 0)),
            pl.BlockSpec((KQ_BLK, 128), lambda j, i: (i, j)),
        ],
        out_shape=[
            jax.ShapeDtypeStruct((Q, NG, GSZ), jnp.float32),
            jax.ShapeDtypeStruct((Q, NT * 128), jnp.float32),
        ],
    )(x, X_pad)


def _mlp_body(x_ref, g_ref, w1x_ref, w1g_ref, b1_ref, w2_ref, b2_ref,
              w3_ref, b3_ref, out_ref):
    hp = jax.lax.Precision.HIGHEST
    a1 = (jax.lax.dot_general(x_ref[...], w1x_ref[...], (((1,), (0,)), ((), ())),
                              preferred_element_type=jnp.float32, precision=hp)
          + jax.lax.dot_general(g_ref[...], w1g_ref[...], (((1,), (0,)), ((), ())),
                                preferred_element_type=jnp.float32, precision=hp)
          + b1_ref[...])
    h1 = jnp.tanh(a1)
    a2 = jax.lax.dot_general(h1, w2_ref[...], (((1,), (0,)), ((), ())),
                             preferred_element_type=jnp.float32, precision=hp) + b2_ref[...]
    h2 = jnp.tanh(a2)
    a3 = jax.lax.dot_general(h2, w3_ref[...], (((1,), (0,)), ((), ())),
                             preferred_element_type=jnp.float32, precision=hp) + b3_ref[0, 0]
    out_ref[...] = jax.nn.sigmoid(a3)


def _mlp(x, g, w1x_t, w1g_t, b1, w2_t, b2, w3_t, b3):
    gin = NN * NF
    return pl.pallas_call(
        _mlp_body,
        grid=(Q // MQ_BLK,),
        in_specs=[
            pl.BlockSpec((MQ_BLK, D), lambda i: (i, 0)),
            pl.BlockSpec((MQ_BLK, gin), lambda i: (i, 0)),
            pl.BlockSpec((D, HIDDEN), lambda i: (0, 0)),
            pl.BlockSpec((gin, HIDDEN), lambda i: (0, 0)),
            pl.BlockSpec((1, HIDDEN), lambda i: (0, 0)),
            pl.BlockSpec((HIDDEN, HIDDEN), lambda i: (0, 0)),
            pl.BlockSpec((1, HIDDEN), lambda i: (0, 0)),
            pl.BlockSpec((HIDDEN, 1), lambda i: (0, 0)),
            pl.BlockSpec((1, 1), lambda i: (0, 0)),
        ],
        out_specs=pl.BlockSpec((MQ_BLK, 1), lambda i: (i, 0)),
        out_shape=jax.ShapeDtypeStruct((Q, 1), jnp.float32),
    )(x, g, w1x_t, w1g_t, b1[None, :], w2_t, b2[None, :], w3_t, b3[None, :])


NSEL = NN + 1
CAP = 128
GRP_A = 7
VB = CAP + 128
VB16 = VB // 16
SPAD = 48
N_WORKERS = 32
QPW = Q // N_WORKERS
INF = np.float32(np.inf)


def _topk_gather(d2g, bmin, xaug):
    mesh = plsc.VectorSubcoreMesh(core_axis_name="c", subcore_axis_name="s")

    @functools.partial(
        pl.kernel,
        out_type=jax.ShapeDtypeStruct((Q, NN, NF), jnp.float32),
        mesh=mesh,
        compiler_params=pltpu.CompilerParams(needs_layout_passes=False),
        scratch_types=[
            pltpu.VMEM((NT * 128,), jnp.float32),
            pltpu.VMEM((NG + 16,), jnp.int32),
            pltpu.VMEM((16, GSZ), jnp.float32),
            pltpu.VMEM((16 * GSZ,), jnp.float32),
            pltpu.VMEM((16,), jnp.int32),
            pltpu.VMEM((VB,), jnp.float32),
            pltpu.VMEM((VB,), jnp.int32),
            pltpu.VMEM((SPAD,), jnp.float32),
            pltpu.VMEM((SPAD,), jnp.int32),
            pltpu.VMEM((NN,), jnp.int32),
            pltpu.VMEM((NN, NF), jnp.float32),
            pltpu.SemaphoreType.DMA,
            pltpu.SemaphoreType.DMA,
        ],
    )
    def sc_kernel(d2g_hbm, bmin_hbm, xaug_hbm, g_hbm, bmrow, gidbuf, fbuf,
                  fflat, idxf, vbuf, ibuf, topv, topi, idxg, grows, fsem,
                  sem):
        wid = lax.axis_index("s") * 2 + lax.axis_index("c")
        lane = lax.iota(jnp.int32, 16)

        def rebuild(count, t):
            del count, t

            def round_body(r, _):
                bestv = vbuf[pl.ds(0, 16)]
                for b in range(1, VB16):
                    bestv = jnp.minimum(bestv, vbuf[pl.ds(b * 16, 16)])
                mv = jnp.min(bestv)
                taken = np.int32(0)
                rr = jnp.full((16,), r, jnp.int32)
                for b in range(VB16):
                    v = vbuf[pl.ds(b * 16, 16)]
                    m = v == mv
                    mi = m.astype(jnp.int32)
                    pre = plsc.cumsum(mi)
                    sel = m & (pre == 1) & (taken == 0)
                    iv = ibuf[pl.ds(b * 16, 16)]
                    plsc.store_scatter(topv, [rr], v, mask=sel)
                    plsc.store_scatter(topi, [rr], iv, mask=sel)
                    vbuf[pl.ds(b * 16, 16)] = jnp.where(sel, INF, v)
                    taken = taken + jnp.sum(mi)
                return mv

            t33 = lax.fori_loop(0, NSEL, round_body, np.float32(0.0))
            for b in range(VB16):
                pos = lane + b * 16
                if b * 16 < SPAD:
                    tv = topv[pl.ds(b * 16, 16)]
                    ti = topi[pl.ds(b * 16, 16)]
                    vbuf[pl.ds(b * 16, 16)] = jnp.where(pos < NSEL, tv, INF)
                    ibuf[pl.ds(b * 16, 16)] = ti
                else:
                    vbuf[pl.ds(b * 16, 16)] = jnp.full((16,), INF)
            return jnp.full((), NSEL, jnp.int32), t33

        zero16 = jnp.zeros((16,), jnp.int32)

        def a_append(i, carry):
            count, t = carry
            v = bmrow[pl.ds(i * 128, 16)]
            m = v < t
            iv = lane + i * 16
            plsc.store_compressed(vbuf.at[pl.ds(count, 16)], v, mask=m)
            plsc.store_compressed(ibuf.at[pl.ds(count, 16)], iv, mask=m)
            return count + jnp.sum(m.astype(jnp.int32)), t

        def a_group(gi, carry):
            count, t = carry
            any_m = jnp.zeros((16,), jnp.bool_)
            for u in range(GRP_A):
                v = bmrow[pl.ds((gi * GRP_A + u) * 128, 16)]
                any_m = any_m | (v < t)

            def slow(c, tt):
                c, tt = lax.fori_loop(gi * GRP_A, gi * GRP_A + GRP_A,
                                      a_append, (c, tt))
                return lax.cond(c >= CAP, rebuild,
                                lambda c2, t2: (c2, t2), c, tt)

            return lax.cond(jnp.any(any_m), slow,
                            lambda c2, t2: (c2, t2), count, t)

        def process_query(jq, _):
            q = wid * QPW + jq
            pltpu.sync_copy(bmin_hbm.at[q], bmrow)
            for b in range(VB16):
                vbuf[pl.ds(b * 16, 16)] = jnp.full((16,), INF)
            count, t = lax.fori_loop(0, NT // GRP_A, a_group,
                                     (np.int32(0), INF))
            _, t0 = rebuild(count, t)

            def a2_chunk(i, cnt):
                bm = bmrow[pl.ds(i * 128, 16)]
                m = bm <= t0
                plsc.store_compressed(gidbuf.at[pl.ds(cnt, 16)],
                                      lane + i * 16, mask=m)
                return cnt + jnp.sum(m.astype(jnp.int32))

            cnt = lax.fori_loop(0, NT, a2_chunk, np.int32(0))

            for b in range(VB16):
                vbuf[pl.ds(b * 16, 16)] = jnp.full((16,), INF)

            def b_batch(b, carry):
                base = b * 16
                gv = gidbuf[pl.ds(base, 16)]
                valid = (lane + base) < cnt
                gv = jnp.where(valid, gv, 0)
                idxf[...] = q * NG + gv
                pltpu.async_copy(d2g_hbm.at[idxf], fbuf, fsem).wait()
                for si in range(16):
                    for jj in range(GSZ // 16):
                        fflat[pl.ds(si * GSZ + jj * 16, 16)] = (
                            fbuf[si, pl.ds(jj * 16, 16)])

                def slot_body(s, carry):
                    count, t = carry
                    pos = base + s
                    gidb = plsc.load_gather(gidbuf, [zero16 + pos])
                    vsv = (zero16 + pos) < cnt
                    any_m = jnp.zeros((16,), jnp.bool_)
                    for jj in range(GSZ // 16):
                        v = fflat[pl.ds(s * GSZ + jj * 16, 16)]
                        any_m = any_m | ((v < t) & vsv)

                    def slow(c, tt):
                        for jj in range(GSZ // 16):
                            v = fflat[pl.ds(s * GSZ + jj * 16, 16)]
                            m = (v < tt) & vsv
                            iv = gidb * GSZ + jj * 16 + lane
                            plsc.store_compressed(vbuf.at[pl.ds(c, 16)],
                                                  v, mask=m)
                            plsc.store_compressed(ibuf.at[pl.ds(c, 16)],
                                                  iv, mask=m)
                            c = c + jnp.sum(m.astype(jnp.int32))
                        return lax.cond(c >= CAP, rebuild,
                                        lambda c2, t2: (c2, t2), c, tt)

                    return lax.cond(jnp.any(any_m), slow,
                                    lambda c2, t2: (c2, t2), count, t)

                return lax.fori_loop(0, 16, slot_body, carry)

            nbatch = (cnt + 15) // 16
            count, t = lax.fori_loop(0, nbatch, b_batch,
                                     (np.int32(0), INF))
            rebuild(count, t)
            idxg[pl.ds(0, 16)] = topi[pl.ds(1, 16)]
            idxg[pl.ds(16, 16)] = topi[pl.ds(17, 16)]
            pltpu.async_copy(xaug_hbm.at[idxg], grows, sem).wait()
            pltpu.sync_copy(grows, g_hbm.at[q])
            return 0

        lax.fori_loop(0, QPW, process_query, 0)

    return sc_kernel(d2g, bmin, xaug)


def kernel(x, X_data, y, W1, b1, W2, b2, W3, b3):
    X_pad = jnp.pad(X_data, ((0, K_PAD - K), (0, 0)))
    d2g3, bmin = _d2_matrix(x, X_pad)
    d2g = d2g3.reshape(Q * NG, GSZ)

    Xaug = jnp.concatenate(
        [X_pad, jnp.pad(y, (0, K_PAD - K))[:, None],
         jnp.zeros((K_PAD, NF - D - 1), jnp.float32)], axis=1)

    g = _topk_gather(d2g, bmin, Xaug).reshape(Q, NN * NF)

    w1x_t = W1[:, :D].T
    w1n = W1[:, D:].reshape(HIDDEN, NN, D + 1)
    w1g = jnp.concatenate(
        [w1n, jnp.zeros((HIDDEN, NN, NF - D - 1), jnp.float32)], axis=2)
    w1g_t = w1g.reshape(HIDDEN, NN * NF).T

    return _mlp(x, g, w1x_t, w1g_t, b1, W2.T, b2, W3.T, b3)
